# baseline (speedup 1.0000x reference)
import functools
import jax
import jax.numpy as jnp
from jax import lax
from jax.experimental import pallas as pl
from jax.experimental.pallas import tpu as pltpu
from jax.experimental.pallas import tpu_sc as plsc

N_ROWS = 320000
N_SEG = 10000
D = 128
NC = 2
NS = 16
NW = NC * NS
ROWS_PER_W = N_ROWS // NW
CHUNK = 128
NFULL = ROWS_PER_W // CHUNK
REM = ROWS_PER_W - NFULL * CHUNK
NCHUNK = NFULL + 1
IDS_PAD = 80
NB = 2
ACC_ROWS = N_SEG + 8
ACC_FULL = ACC_ROWS // CHUNK
ACC_REM = ACC_ROWS - ACC_FULL * CHUNK
OUT_FULL = N_SEG // CHUNK
OUT_REM = N_SEG - OUT_FULL * CHUNK

_mesh = plsc.VectorSubcoreMesh(
    core_axis_name="c", subcore_axis_name="s", num_cores=NC, num_subcores=NS)


@functools.partial(
    pl.kernel,
    out_type=jax.ShapeDtypeStruct((NC, N_SEG, D), jnp.float32),
    mesh=_mesh,
    scratch_types=[
        pltpu.VMEM((IDS_PAD, CHUNK), jnp.int32),
        pltpu.VMEM((NB, CHUNK, D), jnp.float32),
        pltpu.VMEM_SHARED((ACC_ROWS, D), jnp.float32),
        pltpu.SemaphoreType.DMA,
        [pltpu.SemaphoreType.DMA] * NB,
        pltpu.SemaphoreType.DMA,
    ],
)
def _seg_sum_sc(data_hbm, ids_hbm, out_hbm, idx_all, rows_v,
                acc_sh, isem, fsems, dsem):
    cid = lax.axis_index("c")
    sid = lax.axis_index("s")
    wid = cid * NS + sid
    base = wid * ROWS_PER_W

    pltpu.async_copy(ids_hbm.at[pl.ds(wid * IDS_PAD, IDS_PAD)], idx_all, isem)

    zero = jnp.zeros((16,), jnp.float32)

    def _zrow(r, _):
        for k in range(D // 16):
            rows_v[0, r, pl.ds(k * 16, 16)] = zero
        return 0

    lax.fori_loop(0, CHUNK, _zrow, 0)

    def _zacc(i, _):
        @pl.when(i % NS == sid)
        def _():
            pltpu.sync_copy(rows_v.at[0], acc_sh.at[pl.ds(i * CHUNK, CHUNK)])
        return 0

    lax.fori_loop(0, ACC_FULL, _zacc, 0)

    @pl.when(sid == ACC_FULL % NS)
    def _():
        pltpu.sync_copy(rows_v.at[0, pl.ds(0, ACC_REM)],
                        acc_sh.at[pl.ds(ACC_FULL * CHUNK, ACC_REM)])

    pltpu.sync_copy(data_hbm.at[pl.ds(base + NFULL * CHUNK, REM)],
                    rows_v.at[0, pl.ds(0, REM)])

    pltpu.make_async_copy(ids_hbm.at[pl.ds(0, IDS_PAD)], idx_all, isem).wait()
    plsc.subcore_barrier()

    pltpu.sync_copy(rows_v.at[0], acc_sh.at[idx_all.at[NFULL]], add=True)

    def _fetch(b, i, sem):
        pltpu.async_copy(data_hbm.at[pl.ds(base + i * CHUNK, CHUNK)],
                         rows_v.at[b], sem)

    for b in range(NB):
        _fetch(b, b, fsems[b])

    @pl.loop(0, NFULL, step=NB)
    def _ring(g):
        for b in range(NB):
            i = g + b
            pltpu.make_async_copy(data_hbm.at[pl.ds(0, CHUNK)],
                                  rows_v.at[b], fsems[b]).wait()
            pltpu.sync_copy(rows_v.at[b], acc_sh.at[idx_all.at[i]], add=True)

            @pl.when(i + NB < NFULL)
            def _():
                _fetch(b, i + NB, fsems[b])

    plsc.subcore_barrier()

    def _dump(i, _):
        @pl.when(i % NS == sid)
        def _():
            pltpu.async_copy(acc_sh.at[pl.ds(i * CHUNK, CHUNK)],
                             out_hbm.at[cid, pl.ds(i * CHUNK, CHUNK)], dsem)
        return 0

    lax.fori_loop(0, OUT_FULL, _dump, 0)

    @pl.when(sid == OUT_FULL % NS)
    def _():
        pltpu.async_copy(acc_sh.at[pl.ds(OUT_FULL * CHUNK, OUT_REM)],
                         out_hbm.at[cid, pl.ds(OUT_FULL * CHUNK, OUT_REM)],
                         dsem)

    def _dump_wait(i, _):
        @pl.when(i % NS == sid)
        def _():
            pltpu.make_async_copy(
                acc_sh.at[pl.ds(i * CHUNK, CHUNK)],
                out_hbm.at[cid, pl.ds(i * CHUNK, CHUNK)], dsem).wait()
        return 0

    lax.fori_loop(0, OUT_FULL, _dump_wait, 0)

    @pl.when(sid == OUT_FULL % NS)
    def _():
        pltpu.make_async_copy(
            acc_sh.at[pl.ds(OUT_FULL * CHUNK, OUT_REM)],
            out_hbm.at[cid, pl.ds(OUT_FULL * CHUNK, OUT_REM)], dsem).wait()


def _add_body(a_ref, b_ref, o_ref):
    o_ref[...] = a_ref[0] + b_ref[0]


def _combine(partials):
    blk = 2000
    return pl.pallas_call(
        _add_body,
        out_shape=jax.ShapeDtypeStruct((N_SEG, D), jnp.float32),
        grid=(N_SEG // blk,),
        in_specs=[
            pl.BlockSpec((1, blk, D), lambda i: (0, i, 0)),
            pl.BlockSpec((1, blk, D), lambda i: (1, i, 0)),
        ],
        out_specs=pl.BlockSpec((blk, D), lambda i: (i, 0)),
    )(partials, partials)


@jax.jit
def kernel(data, segment_ids):
    ids = segment_ids.astype(jnp.int32).reshape(NW, ROWS_PER_W)
    pad = jnp.full((NW, IDS_PAD * CHUNK - ROWS_PER_W), N_SEG, jnp.int32)
    ids_pad = jnp.concatenate([ids, pad], axis=1).reshape(NW * IDS_PAD, CHUNK)
    partials = _seg_sum_sc(data, ids_pad)
    return _combine(partials)


# speedup vs baseline: 1.0161x; 1.0161x over previous
import functools
import jax
import jax.numpy as jnp
from jax import lax
from jax.experimental import pallas as pl
from jax.experimental.pallas import tpu as pltpu
from jax.experimental.pallas import tpu_sc as plsc

N_ROWS = 320000
N_SEG = 10000
D = 128
NC = 2
NS = 16
NW = NC * NS
ROWS_PER_W = N_ROWS // NW
CHUNK = 128
NFULL = ROWS_PER_W // CHUNK
REM = ROWS_PER_W - NFULL * CHUNK
NB = 2
ACC_FULL = N_SEG // CHUNK
ACC_REM = N_SEG - ACC_FULL * CHUNK

_mesh = plsc.VectorSubcoreMesh(
    core_axis_name="c", subcore_axis_name="s", num_cores=NC, num_subcores=NS)


@functools.partial(
    pl.kernel,
    out_type=jax.ShapeDtypeStruct((NC, N_SEG, D), jnp.float32),
    mesh=_mesh,
    scratch_types=[
        pltpu.VMEM((NB, CHUNK), jnp.int32),
        pltpu.VMEM((NB, CHUNK, D), jnp.float32),
        pltpu.VMEM((REM,), jnp.int32),
        pltpu.VMEM((REM, D), jnp.float32),
        pltpu.VMEM_SHARED((N_SEG, D), jnp.float32),
        pltpu.SemaphoreType.DMA,
        pltpu.SemaphoreType.DMA,
        pltpu.SemaphoreType.DMA,
    ],
)
def _seg_sum_sc(data_hbm, ids_hbm, out_hbm, idx_v, rows_v, idx_r, rows_r,
                acc_sh, sem0, sem1, dsem):
    sems = (sem0, sem1)
    cid = lax.axis_index("c")
    sid = lax.axis_index("s")
    wid = cid * NS + sid

    zero = jnp.zeros((16,), jnp.float32)

    def _zrow(r, _):
        for k in range(D // 16):
            rows_v[0, r, pl.ds(k * 16, 16)] = zero
        return 0

    lax.fori_loop(0, CHUNK, _zrow, 0)

    def _zacc(i, _):
        @pl.when(i % NS == sid)
        def _():
            pltpu.sync_copy(rows_v.at[0], acc_sh.at[pl.ds(i * CHUNK, CHUNK)])
        return 0

    lax.fori_loop(0, ACC_FULL, _zacc, 0)

    @pl.when(sid == ACC_FULL % NS)
    def _():
        pltpu.sync_copy(rows_v.at[0, pl.ds(0, ACC_REM)],
                        acc_sh.at[pl.ds(ACC_FULL * CHUNK, ACC_REM)])

    base = wid * ROWS_PER_W

    def _fetch(b, i, sem):
        off = base + i * CHUNK
        pltpu.async_copy(ids_hbm.at[pl.ds(off, CHUNK)], idx_v.at[b], sem)
        pltpu.async_copy(data_hbm.at[pl.ds(off, CHUNK)], rows_v.at[b], sem)

    def _wait_fetch(b, sem):
        pltpu.make_async_copy(ids_hbm.at[pl.ds(0, CHUNK)], idx_v.at[b],
                              sem).wait()
        pltpu.make_async_copy(data_hbm.at[pl.ds(0, CHUNK)], rows_v.at[b],
                              sem).wait()

    for b in range(NB):
        _fetch(b, b, sems[b])

    plsc.subcore_barrier()

    @pl.loop(0, NFULL, step=NB)
    def _ring(g):
        for b in range(NB):
            i = g + b
            _wait_fetch(b, sems[b])
            pltpu.sync_copy(rows_v.at[b], acc_sh.at[idx_v.at[b]], add=True)

            @pl.when(i + NB < NFULL)
            def _():
                _fetch(b, i + NB, sems[b])

    off = base + NFULL * CHUNK
    pltpu.sync_copy(ids_hbm.at[pl.ds(off, REM)], idx_r)
    pltpu.sync_copy(data_hbm.at[pl.ds(off, REM)], rows_r)
    pltpu.sync_copy(rows_r, acc_sh.at[idx_r], add=True)

    plsc.subcore_barrier()

    def _dump(i, _):
        @pl.when(i % NS == sid)
        def _():
            pltpu.async_copy(acc_sh.at[pl.ds(i * CHUNK, CHUNK)],
                             out_hbm.at[cid, pl.ds(i * CHUNK, CHUNK)], dsem)
        return 0

    lax.fori_loop(0, ACC_FULL, _dump, 0)

    @pl.when(sid == ACC_FULL % NS)
    def _():
        pltpu.async_copy(acc_sh.at[pl.ds(ACC_FULL * CHUNK, ACC_REM)],
                         out_hbm.at[cid, pl.ds(ACC_FULL * CHUNK, ACC_REM)],
                         dsem)

    def _dump_wait(i, _):
        @pl.when(i % NS == sid)
        def _():
            pltpu.make_async_copy(
                acc_sh.at[pl.ds(i * CHUNK, CHUNK)],
                out_hbm.at[cid, pl.ds(i * CHUNK, CHUNK)], dsem).wait()
        return 0

    lax.fori_loop(0, ACC_FULL, _dump_wait, 0)

    @pl.when(sid == ACC_FULL % NS)
    def _():
        pltpu.make_async_copy(
            acc_sh.at[pl.ds(ACC_FULL * CHUNK, ACC_REM)],
            out_hbm.at[cid, pl.ds(ACC_FULL * CHUNK, ACC_REM)], dsem).wait()


def _add_body(a_ref, b_ref, o_ref):
    o_ref[...] = a_ref[0] + b_ref[0]


def _combine(partials):
    blk = 2000
    return pl.pallas_call(
        _add_body,
        out_shape=jax.ShapeDtypeStruct((N_SEG, D), jnp.float32),
        grid=(N_SEG // blk,),
        in_specs=[
            pl.BlockSpec((1, blk, D), lambda i: (0, i, 0)),
            pl.BlockSpec((1, blk, D), lambda i: (1, i, 0)),
        ],
        out_specs=pl.BlockSpec((blk, D), lambda i: (i, 0)),
    )(partials, partials)


@jax.jit
def kernel(data, segment_ids):
    ids = segment_ids.astype(jnp.int32)
    partials = _seg_sum_sc(data, ids)
    return _combine(partials)


# speedup vs baseline: 1.0645x; 1.0476x over previous
import functools
import jax
import jax.numpy as jnp
from jax import lax
from jax.experimental import pallas as pl
from jax.experimental.pallas import tpu as pltpu
from jax.experimental.pallas import tpu_sc as plsc

N_ROWS = 320000
N_SEG = 10000
D = 128
NC = 2
NS = 16
NW = NC * NS
ROWS_PER_W = N_ROWS // NW
CHUNK = 128
NFULL = ROWS_PER_W // CHUNK
REM = ROWS_PER_W - NFULL * CHUNK
NB = 3
ACC_FULL = N_SEG // CHUNK
ACC_REM = N_SEG - ACC_FULL * CHUNK

_mesh = plsc.VectorSubcoreMesh(
    core_axis_name="c", subcore_axis_name="s", num_cores=NC, num_subcores=NS)


@functools.partial(
    pl.kernel,
    out_type=jax.ShapeDtypeStruct((NC, N_SEG, D), jnp.float32),
    mesh=_mesh,
    scratch_types=[
        pltpu.VMEM((NB, CHUNK), jnp.int32),
        pltpu.VMEM((NB, CHUNK, D), jnp.float32),
        pltpu.VMEM((REM,), jnp.int32),
        pltpu.VMEM_SHARED((N_SEG, D), jnp.float32),
        pltpu.SemaphoreType.DMA,
        pltpu.SemaphoreType.DMA,
        pltpu.SemaphoreType.DMA,
        pltpu.SemaphoreType.DMA,
    ],
)
def _seg_sum_sc(data_hbm, ids_hbm, out_hbm, idx_v, rows_v, idx_r,
                acc_sh, sem0, sem1, sem2, dsem):
    sems = (sem0, sem1, sem2)
    cid = lax.axis_index("c")
    sid = lax.axis_index("s")
    wid = cid * NS + sid

    zero = jnp.zeros((16,), jnp.float32)

    def _zrow(r, _):
        for k in range(D // 16):
            rows_v[0, r, pl.ds(k * 16, 16)] = zero
        return 0

    lax.fori_loop(0, CHUNK, _zrow, 0)

    def _zacc(i, _):
        @pl.when(i % NS == sid)
        def _():
            pltpu.sync_copy(rows_v.at[0], acc_sh.at[pl.ds(i * CHUNK, CHUNK)])
        return 0

    lax.fori_loop(0, ACC_FULL, _zacc, 0)

    @pl.when(sid == ACC_FULL % NS)
    def _():
        pltpu.sync_copy(rows_v.at[0, pl.ds(0, ACC_REM)],
                        acc_sh.at[pl.ds(ACC_FULL * CHUNK, ACC_REM)])

    base = wid * ROWS_PER_W
    toff = base + NFULL * CHUNK
    pltpu.sync_copy(ids_hbm.at[pl.ds(toff, REM)], idx_r)
    pltpu.sync_copy(data_hbm.at[pl.ds(toff, REM)],
                    rows_v.at[0, pl.ds(0, REM)])

    def _fetch(b, i, sem):
        off = base + i * CHUNK
        pltpu.async_copy(ids_hbm.at[pl.ds(off, CHUNK)], idx_v.at[b], sem)
        pltpu.async_copy(data_hbm.at[pl.ds(off, CHUNK)], rows_v.at[b], sem)

    def _wait_fetch(b, sem):
        pltpu.make_async_copy(ids_hbm.at[pl.ds(0, CHUNK)], idx_v.at[b],
                              sem).wait()
        pltpu.make_async_copy(data_hbm.at[pl.ds(0, CHUNK)], rows_v.at[b],
                              sem).wait()

    plsc.subcore_barrier()

    pltpu.sync_copy(rows_v.at[0, pl.ds(0, REM)], acc_sh.at[idx_r], add=True)

    for b in range(NB):
        _fetch(b, b, sems[b])

    @pl.loop(0, NFULL, step=NB)
    def _ring(g):
        for b in range(NB):
            i = g + b
            _wait_fetch(b, sems[b])
            pltpu.sync_copy(rows_v.at[b], acc_sh.at[idx_v.at[b]], add=True)

            @pl.when(i + NB < NFULL)
            def _():
                _fetch(b, i + NB, sems[b])

    plsc.subcore_barrier()

    def _dump(i, _):
        @pl.when(i % NS == sid)
        def _():
            pltpu.async_copy(acc_sh.at[pl.ds(i * CHUNK, CHUNK)],
                             out_hbm.at[cid, pl.ds(i * CHUNK, CHUNK)], dsem)
        return 0

    lax.fori_loop(0, ACC_FULL, _dump, 0)

    @pl.when(sid == ACC_FULL % NS)
    def _():
        pltpu.async_copy(acc_sh.at[pl.ds(ACC_FULL * CHUNK, ACC_REM)],
                         out_hbm.at[cid, pl.ds(ACC_FULL * CHUNK, ACC_REM)],
                         dsem)

    def _dump_wait(i, _):
        @pl.when(i % NS == sid)
        def _():
            pltpu.make_async_copy(
                acc_sh.at[pl.ds(i * CHUNK, CHUNK)],
                out_hbm.at[cid, pl.ds(i * CHUNK, CHUNK)], dsem).wait()
        return 0

    lax.fori_loop(0, ACC_FULL, _dump_wait, 0)

    @pl.when(sid == ACC_FULL % NS)
    def _():
        pltpu.make_async_copy(
            acc_sh.at[pl.ds(ACC_FULL * CHUNK, ACC_REM)],
            out_hbm.at[cid, pl.ds(ACC_FULL * CHUNK, ACC_REM)], dsem).wait()


def _add_body(a_ref, b_ref, o_ref):
    o_ref[...] = a_ref[0] + b_ref[0]


def _combine(partials):
    blk = 2000
    return pl.pallas_call(
        _add_body,
        out_shape=jax.ShapeDtypeStruct((N_SEG, D), jnp.float32),
        grid=(N_SEG // blk,),
        in_specs=[
            pl.BlockSpec((1, blk, D), lambda i: (0, i, 0)),
            pl.BlockSpec((1, blk, D), lambda i: (1, i, 0)),
        ],
        out_specs=pl.BlockSpec((blk, D), lambda i: (i, 0)),
    )(partials, partials)


@jax.jit
def kernel(data, segment_ids):
    ids = segment_ids.astype(jnp.int32)
    partials = _seg_sum_sc(data, ids)
    return _combine(partials)


# speedup vs baseline: 1.0804x; 1.0149x over previous
import functools
import jax
import jax.numpy as jnp
from jax import lax
from jax.experimental import pallas as pl
from jax.experimental.pallas import tpu as pltpu
from jax.experimental.pallas import tpu_sc as plsc

N_ROWS = 320000
N_SEG = 10000
D = 128
NC = 2
NS = 16
NW = NC * NS
ROWS_PER_W = N_ROWS // NW
CHUNK = 128
NFULL = ROWS_PER_W // CHUNK
REM = ROWS_PER_W - NFULL * CHUNK
NB = 3
ACC_FULL = N_SEG // CHUNK
ACC_REM = N_SEG - ACC_FULL * CHUNK

_mesh = plsc.VectorSubcoreMesh(
    core_axis_name="c", subcore_axis_name="s", num_cores=NC, num_subcores=NS)


@functools.partial(
    pl.kernel,
    out_type=jax.ShapeDtypeStruct((NC, N_SEG, D), jnp.float32),
    mesh=_mesh,
    scratch_types=[
        pltpu.VMEM((NB, CHUNK), jnp.int32),
        pltpu.VMEM((NB, CHUNK, D), jnp.float32),
        pltpu.VMEM((REM,), jnp.int32),
        pltpu.VMEM_SHARED((N_SEG, D), jnp.float32),
        pltpu.SemaphoreType.DMA,
        pltpu.SemaphoreType.DMA,
        pltpu.SemaphoreType.DMA,
        pltpu.SemaphoreType.DMA,
    ],
)
def _seg_sum_sc(data_hbm, ids_hbm, out_hbm, idx_v, rows_v, idx_r,
                acc_sh, sem0, sem1, sem2, dsem):
    sems = (sem0, sem1, sem2)
    cid = lax.axis_index("c")
    sid = lax.axis_index("s")
    wid = cid * NS + sid

    zero = jnp.zeros((16,), jnp.float32)

    def _zrow(r, _):
        for k in range(D // 16):
            rows_v[0, r, pl.ds(k * 16, 16)] = zero
        return 0

    lax.fori_loop(0, CHUNK, _zrow, 0)

    def _zacc(i, _):
        @pl.when(i % NS == sid)
        def _():
            pltpu.sync_copy(rows_v.at[0], acc_sh.at[pl.ds(i * CHUNK, CHUNK)])
        return 0

    lax.fori_loop(0, ACC_FULL, _zacc, 0)

    @pl.when(sid == ACC_FULL % NS)
    def _():
        pltpu.sync_copy(rows_v.at[0, pl.ds(0, ACC_REM)],
                        acc_sh.at[pl.ds(ACC_FULL * CHUNK, ACC_REM)])

    base = wid * ROWS_PER_W
    toff = base + NFULL * CHUNK
    def _fetch(b, i, sem):
        off = base + i * CHUNK
        pltpu.async_copy(ids_hbm.at[pl.ds(off, CHUNK)], idx_v.at[b], sem)
        pltpu.async_copy(data_hbm.at[pl.ds(off, CHUNK)], rows_v.at[b], sem)

    def _wait_fetch(b, sem):
        pltpu.make_async_copy(ids_hbm.at[pl.ds(0, CHUNK)], idx_v.at[b],
                              sem).wait()
        pltpu.make_async_copy(data_hbm.at[pl.ds(0, CHUNK)], rows_v.at[b],
                              sem).wait()

    for b in range(1, NB):
        _fetch(b, b, sems[b])
    pltpu.sync_copy(ids_hbm.at[pl.ds(toff, REM)], idx_r)
    pltpu.sync_copy(data_hbm.at[pl.ds(toff, REM)],
                    rows_v.at[0, pl.ds(0, REM)])

    plsc.subcore_barrier()

    pltpu.sync_copy(rows_v.at[0, pl.ds(0, REM)], acc_sh.at[idx_r], add=True)

    _fetch(0, 0, sems[0])

    @pl.loop(0, NFULL, step=NB)
    def _ring(g):
        for b in range(NB):
            i = g + b
            _wait_fetch(b, sems[b])
            pltpu.sync_copy(rows_v.at[b], acc_sh.at[idx_v.at[b]], add=True)

            @pl.when(i + NB < NFULL)
            def _():
                _fetch(b, i + NB, sems[b])

    plsc.subcore_barrier()

    def _dump(i, _):
        @pl.when(i % NS == sid)
        def _():
            pltpu.async_copy(acc_sh.at[pl.ds(i * CHUNK, CHUNK)],
                             out_hbm.at[cid, pl.ds(i * CHUNK, CHUNK)], dsem)
        return 0

    lax.fori_loop(0, ACC_FULL, _dump, 0)

    @pl.when(sid == ACC_FULL % NS)
    def _():
        pltpu.async_copy(acc_sh.at[pl.ds(ACC_FULL * CHUNK, ACC_REM)],
                         out_hbm.at[cid, pl.ds(ACC_FULL * CHUNK, ACC_REM)],
                         dsem)

    def _dump_wait(i, _):
        @pl.when(i % NS == sid)
        def _():
            pltpu.make_async_copy(
                acc_sh.at[pl.ds(i * CHUNK, CHUNK)],
                out_hbm.at[cid, pl.ds(i * CHUNK, CHUNK)], dsem).wait()
        return 0

    lax.fori_loop(0, ACC_FULL, _dump_wait, 0)

    @pl.when(sid == ACC_FULL % NS)
    def _():
        pltpu.make_async_copy(
            acc_sh.at[pl.ds(ACC_FULL * CHUNK, ACC_REM)],
            out_hbm.at[cid, pl.ds(ACC_FULL * CHUNK, ACC_REM)], dsem).wait()


def _add_body(a_ref, b_ref, o_ref):
    o_ref[...] = a_ref[0] + b_ref[0]


def _combine(partials):
    blk = 5000
    return pl.pallas_call(
        _add_body,
        out_shape=jax.ShapeDtypeStruct((N_SEG, D), jnp.float32),
        grid=(N_SEG // blk,),
        in_specs=[
            pl.BlockSpec((1, blk, D), lambda i: (0, i, 0)),
            pl.BlockSpec((1, blk, D), lambda i: (1, i, 0)),
        ],
        out_specs=pl.BlockSpec((blk, D), lambda i: (i, 0)),
    )(partials, partials)


@jax.jit
def kernel(data, segment_ids):
    ids = segment_ids.astype(jnp.int32)
    partials = _seg_sum_sc(data, ids)
    return _combine(partials)


# speedup vs baseline: 1.0889x; 1.0079x over previous
import functools
import jax
import jax.numpy as jnp
from jax import lax
from jax.experimental import pallas as pl
from jax.experimental.pallas import tpu as pltpu
from jax.experimental.pallas import tpu_sc as plsc

N_ROWS = 320000
N_SEG = 10000
D = 128
NC = 2
NS = 16
NW = NC * NS
ROWS_PER_W = N_ROWS // NW
CHUNK = 64
NFULL = ROWS_PER_W // CHUNK
REM = ROWS_PER_W - NFULL * CHUNK
NB = 6
ACC_FULL = N_SEG // CHUNK
ACC_REM = N_SEG - ACC_FULL * CHUNK

_mesh = plsc.VectorSubcoreMesh(
    core_axis_name="c", subcore_axis_name="s", num_cores=NC, num_subcores=NS)


@functools.partial(
    pl.kernel,
    out_type=jax.ShapeDtypeStruct((NC, N_SEG, D), jnp.float32),
    mesh=_mesh,
    scratch_types=[
        pltpu.VMEM((NB, CHUNK), jnp.int32),
        pltpu.VMEM((NB, CHUNK, D), jnp.float32),
        pltpu.VMEM((REM,), jnp.int32),
        pltpu.VMEM_SHARED((N_SEG, D), jnp.float32),
        [pltpu.SemaphoreType.DMA] * NB,
        pltpu.SemaphoreType.DMA,
    ],
)
def _seg_sum_sc(data_hbm, ids_hbm, out_hbm, idx_v, rows_v, idx_r,
                acc_sh, sems, dsem):
    cid = lax.axis_index("c")
    sid = lax.axis_index("s")
    wid = cid * NS + sid

    zero = jnp.zeros((16,), jnp.float32)

    def _zrow(r, _):
        for k in range(D // 16):
            rows_v[0, r, pl.ds(k * 16, 16)] = zero
        return 0

    lax.fori_loop(0, CHUNK, _zrow, 0)

    def _zacc(i, _):
        @pl.when(i % NS == sid)
        def _():
            pltpu.sync_copy(rows_v.at[0], acc_sh.at[pl.ds(i * CHUNK, CHUNK)])
        return 0

    lax.fori_loop(0, ACC_FULL, _zacc, 0)

    @pl.when(sid == ACC_FULL % NS)
    def _():
        pltpu.sync_copy(rows_v.at[0, pl.ds(0, ACC_REM)],
                        acc_sh.at[pl.ds(ACC_FULL * CHUNK, ACC_REM)])

    base = wid * ROWS_PER_W
    toff = base + NFULL * CHUNK
    def _fetch(b, i, sem):
        off = base + i * CHUNK
        pltpu.async_copy(ids_hbm.at[pl.ds(off, CHUNK)], idx_v.at[b], sem)
        pltpu.async_copy(data_hbm.at[pl.ds(off, CHUNK)], rows_v.at[b], sem)

    def _wait_fetch(b, sem):
        pltpu.make_async_copy(ids_hbm.at[pl.ds(0, CHUNK)], idx_v.at[b],
                              sem).wait()
        pltpu.make_async_copy(data_hbm.at[pl.ds(0, CHUNK)], rows_v.at[b],
                              sem).wait()

    for b in range(1, NB):
        _fetch(b, b, sems[b])
    pltpu.sync_copy(ids_hbm.at[pl.ds(toff, REM)], idx_r)
    pltpu.sync_copy(data_hbm.at[pl.ds(toff, REM)],
                    rows_v.at[0, pl.ds(0, REM)])

    plsc.subcore_barrier()

    pltpu.sync_copy(rows_v.at[0, pl.ds(0, REM)], acc_sh.at[idx_r], add=True)

    _fetch(0, 0, sems[0])

    @pl.loop(0, NFULL, step=NB)
    def _ring(g):
        for b in range(NB):
            i = g + b
            _wait_fetch(b, sems[b])
            pltpu.sync_copy(rows_v.at[b], acc_sh.at[idx_v.at[b]], add=True)

            @pl.when(i + NB < NFULL)
            def _():
                _fetch(b, i + NB, sems[b])

    plsc.subcore_barrier()

    def _dump(i, _):
        @pl.when(i % NS == sid)
        def _():
            pltpu.async_copy(acc_sh.at[pl.ds(i * CHUNK, CHUNK)],
                             out_hbm.at[cid, pl.ds(i * CHUNK, CHUNK)], dsem)
        return 0

    lax.fori_loop(0, ACC_FULL, _dump, 0)

    @pl.when(sid == ACC_FULL % NS)
    def _():
        pltpu.async_copy(acc_sh.at[pl.ds(ACC_FULL * CHUNK, ACC_REM)],
                         out_hbm.at[cid, pl.ds(ACC_FULL * CHUNK, ACC_REM)],
                         dsem)

    def _dump_wait(i, _):
        @pl.when(i % NS == sid)
        def _():
            pltpu.make_async_copy(
                acc_sh.at[pl.ds(i * CHUNK, CHUNK)],
                out_hbm.at[cid, pl.ds(i * CHUNK, CHUNK)], dsem).wait()
        return 0

    lax.fori_loop(0, ACC_FULL, _dump_wait, 0)

    @pl.when(sid == ACC_FULL % NS)
    def _():
        pltpu.make_async_copy(
            acc_sh.at[pl.ds(ACC_FULL * CHUNK, ACC_REM)],
            out_hbm.at[cid, pl.ds(ACC_FULL * CHUNK, ACC_REM)], dsem).wait()


def _add_body(a_ref, b_ref, o_ref):
    o_ref[...] = a_ref[0] + b_ref[0]


def _combine(partials):
    blk = 5000
    return pl.pallas_call(
        _add_body,
        out_shape=jax.ShapeDtypeStruct((N_SEG, D), jnp.float32),
        grid=(N_SEG // blk,),
        in_specs=[
            pl.BlockSpec((1, blk, D), lambda i: (0, i, 0)),
            pl.BlockSpec((1, blk, D), lambda i: (1, i, 0)),
        ],
        out_specs=pl.BlockSpec((blk, D), lambda i: (i, 0)),
    )(partials, partials)


@jax.jit
def kernel(data, segment_ids):
    ids = segment_ids.astype(jnp.int32)
    partials = _seg_sum_sc(data, ids)
    return _combine(partials)


# speedup vs baseline: 1.0933x; 1.0040x over previous
import functools
import jax
import jax.numpy as jnp
from jax import lax
from jax.experimental import pallas as pl
from jax.experimental.pallas import tpu as pltpu
from jax.experimental.pallas import tpu_sc as plsc

N_ROWS = 320000
N_SEG = 10000
D = 128
NC = 2
NS = 16
NW = NC * NS
ROWS_PER_W = N_ROWS // NW
CHUNK = 40
NFULL = ROWS_PER_W // CHUNK
REM = ROWS_PER_W - NFULL * CHUNK
NB = 5
ACC_FULL = N_SEG // CHUNK
ACC_REM = N_SEG - ACC_FULL * CHUNK

_mesh = plsc.VectorSubcoreMesh(
    core_axis_name="c", subcore_axis_name="s", num_cores=NC, num_subcores=NS)


@functools.partial(
    pl.kernel,
    out_type=jax.ShapeDtypeStruct((NC, N_SEG, D), jnp.float32),
    mesh=_mesh,
    scratch_types=[
        pltpu.VMEM((NB, CHUNK), jnp.int32),
        pltpu.VMEM((NB, CHUNK, D), jnp.float32),
        pltpu.VMEM_SHARED((N_SEG, D), jnp.float32),
        [pltpu.SemaphoreType.DMA] * NB,
        pltpu.SemaphoreType.DMA,
    ],
)
def _seg_sum_sc(data_hbm, ids_hbm, out_hbm, idx_v, rows_v,
                acc_sh, sems, dsem):
    cid = lax.axis_index("c")
    sid = lax.axis_index("s")
    wid = cid * NS + sid

    zero = jnp.zeros((16,), jnp.float32)

    def _zrow(r, _):
        for k in range(D // 16):
            rows_v[0, r, pl.ds(k * 16, 16)] = zero
        return 0

    lax.fori_loop(0, CHUNK, _zrow, 0)

    def _zacc(i, _):
        @pl.when(i % NS == sid)
        def _():
            pltpu.sync_copy(rows_v.at[0], acc_sh.at[pl.ds(i * CHUNK, CHUNK)])
        return 0

    lax.fori_loop(0, ACC_FULL, _zacc, 0)

    if ACC_REM:
        @pl.when(sid == ACC_FULL % NS)
        def _():
            pltpu.sync_copy(rows_v.at[0, pl.ds(0, ACC_REM)],
                            acc_sh.at[pl.ds(ACC_FULL * CHUNK, ACC_REM)])

    base = wid * ROWS_PER_W
    def _fetch(b, i, sem):
        off = base + i * CHUNK
        pltpu.async_copy(ids_hbm.at[pl.ds(off, CHUNK)], idx_v.at[b], sem)
        pltpu.async_copy(data_hbm.at[pl.ds(off, CHUNK)], rows_v.at[b], sem)

    def _wait_fetch(b, sem):
        pltpu.make_async_copy(ids_hbm.at[pl.ds(0, CHUNK)], idx_v.at[b],
                              sem).wait()
        pltpu.make_async_copy(data_hbm.at[pl.ds(0, CHUNK)], rows_v.at[b],
                              sem).wait()

    for b in range(1, NB):
        _fetch(b, b, sems[b])

    plsc.subcore_barrier()

    _fetch(0, 0, sems[0])

    @pl.loop(0, NFULL, step=NB)
    def _ring(g):
        for b in range(NB):
            i = g + b
            _wait_fetch(b, sems[b])
            pltpu.sync_copy(rows_v.at[b], acc_sh.at[idx_v.at[b]], add=True)

            @pl.when(i + NB < NFULL)
            def _():
                _fetch(b, i + NB, sems[b])

    plsc.subcore_barrier()

    def _dump(i, _):
        @pl.when(i % NS == sid)
        def _():
            pltpu.async_copy(acc_sh.at[pl.ds(i * CHUNK, CHUNK)],
                             out_hbm.at[cid, pl.ds(i * CHUNK, CHUNK)], dsem)
        return 0

    lax.fori_loop(0, ACC_FULL, _dump, 0)

    if ACC_REM:
        @pl.when(sid == ACC_FULL % NS)
        def _():
            pltpu.async_copy(acc_sh.at[pl.ds(ACC_FULL * CHUNK, ACC_REM)],
                             out_hbm.at[cid, pl.ds(ACC_FULL * CHUNK, ACC_REM)],
                             dsem)

    def _dump_wait(i, _):
        @pl.when(i % NS == sid)
        def _():
            pltpu.make_async_copy(
                acc_sh.at[pl.ds(i * CHUNK, CHUNK)],
                out_hbm.at[cid, pl.ds(i * CHUNK, CHUNK)], dsem).wait()
        return 0

    lax.fori_loop(0, ACC_FULL, _dump_wait, 0)

    if ACC_REM:
        @pl.when(sid == ACC_FULL % NS)
        def _():
            pltpu.make_async_copy(
                acc_sh.at[pl.ds(ACC_FULL * CHUNK, ACC_REM)],
                out_hbm.at[cid, pl.ds(ACC_FULL * CHUNK, ACC_REM)], dsem).wait()


def _add_body(a_ref, b_ref, o_ref):
    o_ref[...] = a_ref[0] + b_ref[0]


def _combine(partials):
    blk = 5000
    return pl.pallas_call(
        _add_body,
        out_shape=jax.ShapeDtypeStruct((N_SEG, D), jnp.float32),
        grid=(N_SEG // blk,),
        in_specs=[
            pl.BlockSpec((1, blk, D), lambda i: (0, i, 0)),
            pl.BlockSpec((1, blk, D), lambda i: (1, i, 0)),
        ],
        out_specs=pl.BlockSpec((blk, D), lambda i: (i, 0)),
    )(partials, partials)


@jax.jit
def kernel(data, segment_ids):
    ids = segment_ids.astype(jnp.int32)
    partials = _seg_sum_sc(data, ids)
    return _combine(partials)
